# baseline (speedup 1.0000x reference)
import functools

import jax
import jax.numpy as jnp
from jax import lax
from jax.experimental import pallas as pl
from jax.experimental.pallas import tpu as pltpu
from jax.experimental.pallas import tpu_sc as plsc

P, C, D = 5, 100000, 64
B, T = 4096, 200
N = B * T

_ROWS = 2048
_NFG = -(-C // _ROWS)
_BG_ROW = _NFG * _ROWS
_TABLE_ROWS = (_NFG + 1) * _ROWS


def _table_body(fg_ref, bg_ref, out_ref):
    j = pl.program_id(0)

    @pl.when(j < _NFG)
    def _fg():
        x = fg_ref[...]
        m = jnp.sum(x, axis=0) * (1.0 / P)
        norm = jnp.sqrt(jnp.sum(m * m, axis=0, keepdims=True))
        normed = (m / jnp.maximum(norm, 1e-5)).T
        out_ref[...] = jnp.concatenate(
            [normed, jnp.zeros((_ROWS, 128 - D), jnp.float32)], axis=1
        )

    @pl.when(j == _NFG)
    def _bg():
        b = bg_ref[...]
        norm = jnp.sqrt(jnp.sum(b * b, axis=1, keepdims=True))
        normed = jnp.concatenate(
            [b / jnp.maximum(norm, 1e-5), jnp.zeros((1, 128 - D), jnp.float32)],
            axis=1,
        )
        out_ref[...] = jnp.broadcast_to(normed, (_ROWS, 128))


def _build_table(fg, bg):
    fg_t = jnp.transpose(fg, (0, 2, 1))
    return pl.pallas_call(
        _table_body,
        grid=(_NFG + 1,),
        in_specs=[
            pl.BlockSpec((P, D, _ROWS), lambda j: (0, 0, jnp.minimum(j, _NFG - 1))),
            pl.BlockSpec((1, D), lambda j: (0, 0)),
        ],
        out_specs=pl.BlockSpec((_ROWS, 128), lambda j: (j, 0)),
        out_shape=jax.ShapeDtypeStruct((_TABLE_ROWS, 128), jnp.float32),
    )(fg_t, bg)


def _remap_body(t_ref, out_ref):
    t = t_ref[...]
    out_ref[...] = jnp.where(t == 0, _BG_ROW, t - 1)


def _remap_indices(transcripts):
    t_t = jnp.transpose(transcripts.astype(jnp.int32))
    return pl.pallas_call(
        _remap_body,
        grid=(8,),
        in_specs=[pl.BlockSpec((T, B // 8), lambda j: (0, j))],
        out_specs=pl.BlockSpec((T, B // 8), lambda j: (0, j)),
        out_shape=jax.ShapeDtypeStruct((T, B), jnp.int32),
    )(t_t)


_NC, _NS = 2, 16
_NW = _NC * _NS
_BPW = B // _NW
_Q = 5
_L = 16


def _gather_body(table_hbm, idx_hbm, out_hbm, idx_v, rows_v, rowsT, gsems, ssems):
    wid = lax.axis_index("s") * _NC + lax.axis_index("c")
    b0 = wid * _BPW
    pltpu.sync_copy(idx_hbm.at[:, pl.ds(b0, _BPW)], idx_v)
    lane = lax.iota(jnp.int32, _L)

    for q in range(_Q):
        pltpu.async_copy(table_hbm.at[idx_v.at[q]], rows_v.at[q], gsems[q])

    def chunk(t, q, p):
        pltpu.make_async_copy(
            table_hbm.at[pl.ds(0, _BPW)], rows_v.at[q], gsems[q]
        ).wait()

        @pl.when(t >= 2)
        def _drain():
            pltpu.make_async_copy(
                rowsT.at[p], out_hbm.at[0, :, pl.ds(b0, _BPW)], ssems[p]
            ).wait()

        def dstep(d, carry):
            for g in range(_BPW // _L):
                rows = lane + (g * _L)
                col = jnp.full((_L,), 0, jnp.int32) + d
                val = plsc.load_gather(rows_v.at[q], [rows, col])
                rowsT[p, d, pl.ds(g * _L, _L)] = val
            return carry

        lax.fori_loop(0, D, dstep, 0)
        pltpu.async_copy(rowsT.at[p], out_hbm.at[t, :, pl.ds(b0, _BPW)], ssems[p])

        @pl.when(t + _Q < T)
        def _refill():
            pltpu.async_copy(
                table_hbm.at[idx_v.at[t + _Q]], rows_v.at[q], gsems[q]
            )

    def group(s, carry):
        for qq in range(2 * _Q):
            t = s * (2 * _Q) + qq
            chunk(t, qq % _Q, qq % 2)
        return carry

    lax.fori_loop(0, T // (2 * _Q), group, 0)
    for p in range(2):
        pltpu.make_async_copy(
            rowsT.at[p], out_hbm.at[0, :, pl.ds(b0, _BPW)], ssems[p]
        ).wait()


@functools.cache
def _make_gather():
    @functools.partial(
        pl.kernel,
        mesh=plsc.VectorSubcoreMesh(core_axis_name="c", subcore_axis_name="s"),
        out_type=jax.ShapeDtypeStruct((T, D, B), jnp.float32),
        compiler_params=pltpu.CompilerParams(needs_layout_passes=False),
        scratch_types=[
            pltpu.VMEM((T, _BPW), jnp.int32),
            pltpu.VMEM((_Q, _BPW, 128), jnp.float32),
            pltpu.VMEM((2, D, _BPW), jnp.float32),
            [pltpu.SemaphoreType.DMA] * _Q,
            [pltpu.SemaphoreType.DMA] * 2,
        ],
    )
    def _gather_rows(table_hbm, idx_hbm, out_hbm, idx_v, rows_v, rowsT, gsems, ssems):
        _gather_body(table_hbm, idx_hbm, out_hbm, idx_v, rows_v, rowsT, gsems, ssems)

    return _gather_rows


def kernel(transcripts, fg_action_embedding, bg_embedding):
    table = _build_table(fg_action_embedding, bg_embedding)
    idx = _remap_indices(transcripts)
    out = _make_gather()(table, idx)
    return jnp.transpose(out, (2, 0, 1))
